# Initial kernel scaffold; baseline (speedup 1.0000x reference)
import jax
import jax.numpy as jnp
from jax.experimental import pallas as pl


def kernel(x, edge_index, pv_idx, W1, b1, W2, b2, Wf, bf):
    raise NotImplementedError("write your pallas kernel here")


# speedup vs baseline: 7.0564x; 7.0564x over previous
import functools

import jax
import jax.numpy as jnp
from jax.experimental import pallas as pl
from jax.experimental.pallas import tpu as pltpu
from jax.experimental.pallas import tpu_sc as plsc

N = 10000
D_IN = 128
D_H = 128
N_CLS = 64
NC = 2
NS = 16
NW = NC * NS


def _tc_lin_relu(x, wt, b):
    def body(x_ref, w_ref, b_ref, o_ref):
        o_ref[...] = jnp.maximum(
            jnp.dot(x_ref[...], w_ref[...], preferred_element_type=jnp.float32)
            + b_ref[...], 0.0)
    return pl.pallas_call(
        body,
        out_shape=jax.ShapeDtypeStruct((x.shape[0], wt.shape[1]), jnp.float32),
    )(x, wt, b)


def _tc_add3_lin(y, p0, p1, wt, b, relu_out):
    def body(y_ref, p0_ref, p1_ref, w_ref, b_ref, o_ref):
        h = jnp.maximum(y_ref[...] + p0_ref[...] + p1_ref[...], 0.0)
        o = jnp.dot(h, w_ref[...], preferred_element_type=jnp.float32) + b_ref[...]
        o_ref[...] = jnp.maximum(o, 0.0) if relu_out else o
    return pl.pallas_call(
        body,
        out_shape=jax.ShapeDtypeStruct((y.shape[0], wt.shape[1]), jnp.float32),
    )(y, p0, p1, wt, b)


def _tc_pool_log_softmax(r0, r1, n_cls):
    def body(a_ref, b_ref, o_ref):
        z = a_ref[...][:, :n_cls] + b_ref[...][:, :n_cls]
        z = z - jnp.max(z, axis=1, keepdims=True)
        o_ref[...] = z - jnp.log(jnp.sum(jnp.exp(z), axis=1, keepdims=True))
    return pl.pallas_call(
        body,
        out_shape=jax.ShapeDtypeStruct((r0.shape[0], n_cls), jnp.float32),
    )(r0, r1)


def _sc_edge_scatter(y, src, dst):
    e_tot = src.shape[0]
    d_feat = y.shape[1]
    ew = e_tot // NW
    k = 80
    ch = ew // k
    assert ew * NW == e_tot and ch * k == ew
    nchunks = N // k
    zmax = (nchunks + NS - 1) // NS

    mesh = plsc.VectorSubcoreMesh(core_axis_name="c", subcore_axis_name="s")

    @functools.partial(
        pl.kernel,
        out_type=jax.ShapeDtypeStruct((NC, N, d_feat), jnp.float32),
        mesh=mesh,
        scratch_types=[
            pltpu.VMEM((k,), jnp.int32),
            pltpu.VMEM((k,), jnp.int32),
            pltpu.VMEM((k, d_feat), jnp.float32),
            pltpu.VMEM_SHARED((N, d_feat), jnp.float32),
            pltpu.SemaphoreType.DMA,
        ],
    )
    def kern(y_hbm, src_hbm, dst_hbm, out_hbm, srcv, dstv, rows, acc, sem):
        cid = jax.lax.axis_index("c")
        sid = jax.lax.axis_index("s")
        wid = cid * NS + sid

        lanes = d_feat // 16

        def zbody(t, carry):
            rows[t // lanes, pl.ds((t % lanes) * 16, 16)] = jnp.zeros(
                (16,), jnp.float32)
            return carry
        jax.lax.fori_loop(0, k * lanes, zbody, None)

        for j in range(zmax):
            c = sid * zmax + j

            @pl.when(c < nchunks)
            def _():
                pltpu.sync_copy(rows, acc.at[pl.ds(c * k, k)])

        plsc.subcore_barrier()

        def ebody(i, carry):
            ebase = wid * ew + i * k
            pltpu.sync_copy(src_hbm.at[pl.ds(ebase, k)], srcv)
            pltpu.sync_copy(dst_hbm.at[pl.ds(ebase, k)], dstv)
            pltpu.async_copy(y_hbm.at[srcv], rows, sem).wait()
            pltpu.sync_copy(rows, acc.at[dstv], add=True)
            return carry
        jax.lax.fori_loop(0, ch, ebody, None)
        plsc.subcore_barrier()

        for j in range(zmax):
            c = sid * zmax + j

            @pl.when(c < nchunks)
            def _():
                pltpu.sync_copy(acc.at[pl.ds(c * k, k)],
                                out_hbm.at[cid, pl.ds(c * k, k)])

    return kern(y, src, dst)


def _sc_pv_scatter(logits, pv):
    d_feat = logits.shape[1]
    k = 80
    ch_tot = N // k
    jmax = (ch_tot + NW - 1) // NW
    zmax = (ch_tot + NS - 1) // NS

    mesh = plsc.VectorSubcoreMesh(core_axis_name="c", subcore_axis_name="s")

    @functools.partial(
        pl.kernel,
        out_type=jax.ShapeDtypeStruct((NC, N, d_feat), jnp.float32),
        mesh=mesh,
        scratch_types=[
            pltpu.VMEM((k,), jnp.int32),
            pltpu.VMEM((k, d_feat), jnp.float32),
            pltpu.VMEM_SHARED((N, d_feat), jnp.float32),
        ],
    )
    def kern(l_hbm, pv_hbm, out_hbm, idxv, rows, acc):
        cid = jax.lax.axis_index("c")
        sid = jax.lax.axis_index("s")
        wid = cid * NS + sid

        lanes = d_feat // 16

        def zbody(t, carry):
            rows[t // lanes, pl.ds((t % lanes) * 16, 16)] = jnp.zeros(
                (16,), jnp.float32)
            return carry
        jax.lax.fori_loop(0, k * lanes, zbody, None)

        for j in range(zmax):
            c = sid * zmax + j

            @pl.when(c < ch_tot)
            def _():
                pltpu.sync_copy(rows, acc.at[pl.ds(c * k, k)])

        plsc.subcore_barrier()

        for j in range(jmax):
            c = wid * jmax + j

            @pl.when(c < ch_tot)
            def _():
                rbase = c * k
                pltpu.sync_copy(pv_hbm.at[pl.ds(rbase, k)], idxv)
                pltpu.sync_copy(l_hbm.at[pl.ds(rbase, k)], rows)
                pltpu.sync_copy(rows, acc.at[idxv], add=True)

        plsc.subcore_barrier()

        for j in range(zmax):
            c = sid * zmax + j

            @pl.when(c < ch_tot)
            def _():
                pltpu.sync_copy(acc.at[pl.ds(c * k, k)],
                                out_hbm.at[cid, pl.ds(c * k, k)])

    return kern(logits, pv)


def kernel(x, edge_index, pv_idx, W1, b1, W2, b2, Wf, bf):
    src = edge_index[0].astype(jnp.int32)
    dst = edge_index[1].astype(jnp.int32)
    pv = pv_idx.astype(jnp.int32)
    w1t = W1.T
    w2t = W2.T
    b1r = b1.reshape(1, -1)
    b2r = b2.reshape(1, -1)
    n_cls = Wf.shape[0]
    wft = jnp.pad(Wf.T, ((0, 0), (0, D_H - n_cls)))
    bfr = jnp.pad(bf, (0, D_H - n_cls)).reshape(1, -1)

    y1 = _tc_lin_relu(x, w1t, b1r)
    p = _sc_edge_scatter(y1, src, dst)
    y2 = _tc_add3_lin(y1, p[0], p[1], w2t, b2r, relu_out=True)
    q = _sc_edge_scatter(y2, src, dst)
    logits = _tc_add3_lin(y2, q[0], q[1], wft, bfr, relu_out=False)
    r = _sc_pv_scatter(logits, pv)
    return _tc_pool_log_softmax(r[0], r[1], n_cls)
